# baseline (speedup 1.0000x reference)
import functools

import jax
import jax.numpy as jnp
from jax import lax
from jax.experimental import pallas as pl
from jax.experimental.pallas import tpu as pltpu
from jax.experimental.pallas import tpu_sc as plsc

_N = 10000
_E = 320000
_H = 128
_B = 16
_C = 10
_L = 3

_NC = 2
_NS = 16
_NW = _NC * _NS
_EPW = _E // _NW
_K = 104
_NCHUNK = 98
_EPWP = _NCHUNK * _K
_NACC = _N + 16

_ZROWS = 640
_ZLAST = _N - 15 * _ZROWS


def _seg_body(hm, srcr, dstr, zrows, part, src_v, dst_v, rows0, rows1,
              accum, gs0, gs1, ss0, ss1):
    cid = lax.axis_index("c")
    sid = lax.axis_index("s")
    wid = sid * _NC + cid

    pltpu.sync_copy(srcr.at[wid], src_v)
    pltpu.sync_copy(dstr.at[wid], dst_v)

    base = sid * _ZROWS

    @pl.when(sid < _NS - 1)
    def _():
        pltpu.sync_copy(zrows, accum.at[pl.ds(base, _ZROWS)])

    @pl.when(sid == _NS - 1)
    def _():
        pltpu.sync_copy(zrows.at[pl.ds(0, _ZLAST)],
                        accum.at[pl.ds(15 * _ZROWS, _ZLAST)])

    plsc.subcore_barrier()

    def gather(j, buf, sem):
        pltpu.async_copy(hm.at[src_v.at[pl.ds(j * _K, _K)]], buf, sem)

    def gwait(buf, sem):
        pltpu.make_async_copy(hm.at[src_v.at[pl.ds(0, _K)]], buf, sem).wait()

    def scat(j, buf, sem):
        pltpu.async_copy(buf, accum.at[dst_v.at[j]], sem, add=True)

    def swait(buf, sem):
        pltpu.make_async_copy(hm.at[src_v.at[pl.ds(0, _K)]], buf, sem).wait()

    gather(0, rows0, gs0)
    gather(1, rows1, gs1)

    def pipe(t, carry):
        j = 2 * t
        gwait(rows0, gs0)
        scat(j, rows0, ss0)

        @pl.when(j + 2 < _NCHUNK)
        def _():
            swait(rows0, ss0)
            gather(j + 2, rows0, gs0)

        @pl.when(j + 1 < _NCHUNK)
        def _():
            gwait(rows1, gs1)
            scat(j + 1, rows1, ss1)

        @pl.when(j + 3 < _NCHUNK)
        def _():
            swait(rows1, ss1)
            gather(j + 3, rows1, gs1)

        return carry

    lax.fori_loop(0, (_NCHUNK + 1) // 2, pipe, 0)
    swait(rows0, ss0)
    swait(rows1, ss1)

    plsc.subcore_barrier()

    @pl.when(sid < _NS - 1)
    def _():
        pltpu.sync_copy(accum.at[pl.ds(base, _ZROWS)],
                        part.at[cid, pl.ds(base, _ZROWS)])

    @pl.when(sid == _NS - 1)
    def _():
        pltpu.sync_copy(accum.at[pl.ds(15 * _ZROWS, _ZLAST)],
                        part.at[cid, pl.ds(15 * _ZROWS, _ZLAST)])


@functools.cache
def _get_seg_kernel():
    return pl.kernel(
        _seg_body,
        out_type=jax.ShapeDtypeStruct((_NC, _N, _H), jnp.float32),
        mesh=plsc.VectorSubcoreMesh(core_axis_name="c", subcore_axis_name="s",
                                    num_cores=_NC, num_subcores=_NS),
        scratch_types=[
            pltpu.VMEM((_EPWP,), jnp.int32),
            pltpu.VMEM((_NCHUNK, _K), jnp.int32),
            pltpu.VMEM((_K, _H), jnp.float32),
            pltpu.VMEM((_K, _H), jnp.float32),
            pltpu.VMEM_SHARED((_NACC, _H), jnp.float32),
            pltpu.SemaphoreType.DMA,
            pltpu.SemaphoreType.DMA,
            pltpu.SemaphoreType.DMA,
            pltpu.SemaphoreType.DMA,
        ],
    )

_BLK = 1000
_GRID = _N // _BLK


def _row_spec(r, c):
    return pl.BlockSpec((r, c), lambda i: (i, 0))


def _full_spec(r, c):
    return pl.BlockSpec((r, c), lambda i: (0, 0))


def _lin0_body(x, w, b, o):
    o[...] = jnp.dot(x[...], w[...], preferred_element_type=jnp.float32) + b[...]


_lin0 = pl.pallas_call(
    _lin0_body,
    grid=(_GRID,),
    in_specs=[_row_spec(_BLK, _H), _full_spec(_H, _H), _full_spec(1, _H)],
    out_specs=_row_spec(_BLK, _H),
    out_shape=jax.ShapeDtypeStruct((_N, _H), jnp.float32),
)


def _preb_body(hm, w2, b2, w3b, o):
    bb = jax.nn.relu(jnp.dot(hm[...], w2[...], preferred_element_type=jnp.float32)
                     + b2[...])
    o[...] = jnp.dot(bb, w3b[...], preferred_element_type=jnp.float32)


_preb = pl.pallas_call(
    _preb_body,
    grid=(_GRID,),
    in_specs=[
        _row_spec(_BLK, _H),
        _full_spec(_H, _H), _full_spec(1, _H), _full_spec(_H, 1),
    ],
    out_specs=_row_spec(_BLK, 1),
    out_shape=jax.ShapeDtypeStruct((_N, 1), jnp.float32),
)


def _mask_body(part, h, preb, w1, b1, w3a, b3, mask_o, hm2_o):
    aggr = part[0] + part[1]
    a = jax.nn.relu(jnp.dot(aggr, w1[...], preferred_element_type=jnp.float32)
                    + b1[...])
    logit = (jnp.dot(a, w3a[...], preferred_element_type=jnp.float32)
             + preb[...] + b3[...])
    mask = jax.nn.sigmoid(logit)
    mask_o[...] = mask
    hm2_o[...] = h[...] * mask


_mask_stage = pl.pallas_call(
    _mask_body,
    grid=(_GRID,),
    in_specs=[
        pl.BlockSpec((_NC, _BLK, _H), lambda i: (0, i, 0)),
        _row_spec(_BLK, _H),
        _row_spec(_BLK, 1),
        _full_spec(_H, _H), _full_spec(1, _H),
        _full_spec(_H, 1), _full_spec(1, 1),
    ],
    out_specs=[_row_spec(_BLK, 1), _row_spec(_BLK, _H)],
    out_shape=[
        jax.ShapeDtypeStruct((_N, 1), jnp.float32),
        jax.ShapeDtypeStruct((_N, _H), jnp.float32),
    ],
)


def _prec_body(hm2, w2, b2, o):
    o[...] = jnp.dot(hm2[...], w2[...],
                     preferred_element_type=jnp.float32) + b2[...]


_prec = pl.pallas_call(
    _prec_body,
    grid=(_GRID,),
    in_specs=[_row_spec(_BLK, _H), _full_spec(_H, _H), _full_spec(1, _H)],
    out_specs=_row_spec(_BLK, _H),
    out_shape=jax.ShapeDtypeStruct((_N, _H), jnp.float32),
)


def _conv_body(part, prec, mask, w1, b1, h_o, hmn_o):
    aggr = part[0] + part[1]
    h_new = jax.nn.relu(
        jnp.dot(aggr, w1[...], preferred_element_type=jnp.float32) + b1[...]
        + prec[...])
    h_o[...] = h_new
    hmn_o[...] = h_new * mask[...]


_conv_stage = pl.pallas_call(
    _conv_body,
    grid=(_GRID,),
    in_specs=[
        pl.BlockSpec((_NC, _BLK, _H), lambda i: (0, i, 0)),
        _row_spec(_BLK, _H),
        _row_spec(_BLK, 1),
        _full_spec(_H, _H), _full_spec(1, _H),
    ],
    out_specs=[_row_spec(_BLK, _H), _row_spec(_BLK, _H)],
    out_shape=[
        jax.ShapeDtypeStruct((_N, _H), jnp.float32),
        jax.ShapeDtypeStruct((_N, _H), jnp.float32),
    ],
)


def _head_body(h, batch, w1, b1, w2, b2, o, acc):
    i = pl.program_id(0)

    @pl.when(i == 0)
    def _():
        acc[...] = jnp.zeros_like(acc)

    onehot = (lax.broadcasted_iota(jnp.int32, (_B, _BLK), 0)
              == batch[0]).astype(jnp.float32)
    acc[...] += jnp.dot(onehot, h[...], preferred_element_type=jnp.float32)

    @pl.when(i == _GRID - 1)
    def _():
        pooled = acc[...]
        t = jax.nn.relu(
            jnp.dot(pooled, w1[...], preferred_element_type=jnp.float32) + b1[...])
        z = jnp.dot(t, w2[...], preferred_element_type=jnp.float32) + b2[...]
        m = jnp.max(z, axis=-1, keepdims=True)
        lse = m + jnp.log(jnp.sum(jnp.exp(z - m), axis=-1, keepdims=True))
        o[...] = z - lse


_head = pl.pallas_call(
    _head_body,
    grid=(_GRID,),
    in_specs=[
        _row_spec(_BLK, _H),
        pl.BlockSpec((1, 1, _BLK), lambda i: (i, 0, 0)),
        _full_spec(_H, _H), _full_spec(1, _H),
        _full_spec(_H, _C), _full_spec(1, _C),
    ],
    out_specs=_full_spec(_B, _C),
    out_shape=jax.ShapeDtypeStruct((_B, _C), jnp.float32),
    scratch_shapes=[pltpu.VMEM((_B, _H), jnp.float32)],
)


def kernel(x, edge_index, batch, lin0_W, lin0_b, mask_W1, mask_b1, mask_W2,
           mask_b2, mask_W3, mask_b3, conv_W1, conv_b1, conv_W2, conv_b2,
           lin1_W, lin1_b, lin2_W, lin2_b):
    pad = _EPWP - _EPW
    srcr = jnp.pad(edge_index[0].reshape(_NW, _EPW), ((0, 0), (0, pad)))
    dummy = _N + (jnp.arange(_NW, dtype=jnp.int32) % 16)[:, None]
    dstr = jnp.concatenate(
        [edge_index[1].reshape(_NW, _EPW),
         jnp.broadcast_to(dummy, (_NW, pad))],
        axis=1).reshape(_NW, _NCHUNK, _K)
    zrows = jnp.zeros((_ZROWS, _H), jnp.float32)
    batch3d = batch.reshape(_GRID, 1, _BLK)

    seg = _get_seg_kernel()
    h = _lin0(x, lin0_W, lin0_b.reshape(1, _H))
    hm = h
    for i in range(_L):
        part = seg(hm, srcr, dstr, zrows)
        preb = _preb(hm, mask_W2[i], mask_b2[i].reshape(1, _H),
                     mask_W3[i, _H:])
        mask, hm2 = _mask_stage(
            part, h, preb,
            mask_W1[i], mask_b1[i].reshape(1, _H),
            mask_W3[i, :_H], mask_b3[i].reshape(1, 1))
        part2 = seg(hm2, srcr, dstr, zrows)
        prec = _prec(hm2, conv_W2[i], conv_b2[i].reshape(1, _H))
        h, hm = _conv_stage(
            part2, prec, mask,
            conv_W1[i], conv_b1[i].reshape(1, _H))

    return _head(h, batch3d, lin1_W, lin1_b.reshape(1, _H),
                 lin2_W, lin2_b.reshape(1, _C))


# speedup vs baseline: 1.5749x; 1.5749x over previous
import functools

import jax
import jax.numpy as jnp
from jax import lax
from jax.experimental import pallas as pl
from jax.experimental.pallas import tpu as pltpu
from jax.experimental.pallas import tpu_sc as plsc

_N = 10000
_E = 320000
_H = 128
_B = 16
_C = 10
_L = 3

_NC = 2
_NS = 16
_NW = _NC * _NS
_EPW = _E // _NW
_K = 96
_NCHUNK = 105
_EPWP = _NCHUNK * _K
_NACC = _N + 8

_ZROWS = 640
_ZLAST = _N - 15 * _ZROWS


def _seg_body(hm, srcr, dstr, zrows, part, src_v, dst_v, rows0, rows1,
              accum, gs0, gs1, ss0, ss1):
    cid = lax.axis_index("c")
    sid = lax.axis_index("s")
    wid = sid * _NC + cid

    pltpu.sync_copy(srcr.at[wid], src_v)
    pltpu.sync_copy(dstr.at[wid], dst_v)

    base = sid * _ZROWS

    @pl.when(sid < _NS - 1)
    def _():
        pltpu.sync_copy(zrows, accum.at[pl.ds(base, _ZROWS)])

    @pl.when(sid == _NS - 1)
    def _():
        pltpu.sync_copy(zrows.at[pl.ds(0, _ZLAST)],
                        accum.at[pl.ds(15 * _ZROWS, _ZLAST)])

    plsc.subcore_barrier()

    def gather(j, buf, sem):
        pltpu.async_copy(hm.at[src_v.at[pl.ds(j * _K, _K)]], buf, sem)

    def gwait(buf, sem):
        pltpu.make_async_copy(hm.at[src_v.at[pl.ds(0, _K)]], buf, sem).wait()

    def scat(j, buf, sem):
        pltpu.async_copy(buf, accum.at[dst_v.at[j]], sem, add=True)

    def swait(buf, sem):
        pltpu.make_async_copy(hm.at[src_v.at[pl.ds(0, _K)]], buf, sem).wait()

    gather(0, rows0, gs0)
    gather(1, rows1, gs1)

    def pipe(t, carry):
        j = 2 * t
        gwait(rows0, gs0)
        scat(j, rows0, ss0)

        @pl.when(j + 2 < _NCHUNK)
        def _():
            swait(rows0, ss0)
            gather(j + 2, rows0, gs0)

        @pl.when(j + 1 < _NCHUNK)
        def _():
            gwait(rows1, gs1)
            scat(j + 1, rows1, ss1)

        @pl.when(j + 3 < _NCHUNK)
        def _():
            swait(rows1, ss1)
            gather(j + 3, rows1, gs1)

        return carry

    lax.fori_loop(0, (_NCHUNK + 1) // 2, pipe, 0)
    swait(rows0, ss0)
    swait(rows1, ss1)

    plsc.subcore_barrier()

    @pl.when(sid < _NS - 1)
    def _():
        pltpu.sync_copy(accum.at[pl.ds(base, _ZROWS)],
                        part.at[cid, pl.ds(base, _ZROWS)])

    @pl.when(sid == _NS - 1)
    def _():
        pltpu.sync_copy(accum.at[pl.ds(15 * _ZROWS, _ZLAST)],
                        part.at[cid, pl.ds(15 * _ZROWS, _ZLAST)])


@functools.cache
def _get_seg_kernel():
    return pl.kernel(
        _seg_body,
        out_type=jax.ShapeDtypeStruct((_NC, _N, _H), jnp.float32),
        mesh=plsc.VectorSubcoreMesh(core_axis_name="c", subcore_axis_name="s",
                                    num_cores=_NC, num_subcores=_NS),
        scratch_types=[
            pltpu.VMEM((_EPWP,), jnp.int32),
            pltpu.VMEM((_NCHUNK, _K), jnp.int32),
            pltpu.VMEM((_K, _H), jnp.float32),
            pltpu.VMEM((_K, _H), jnp.float32),
            pltpu.VMEM_SHARED((_NACC, _H), jnp.float32),
            pltpu.SemaphoreType.DMA,
            pltpu.SemaphoreType.DMA,
            pltpu.SemaphoreType.DMA,
            pltpu.SemaphoreType.DMA,
        ],
    )

_BLK = 1000
_GRID = _N // _BLK


def _row_spec(r, c):
    return pl.BlockSpec((r, c), lambda i: (i, 0))


def _full_spec(r, c):
    return pl.BlockSpec((r, c), lambda i: (0, 0))


def _lin0_body(x, w, b, o):
    o[...] = jnp.dot(x[...], w[...], preferred_element_type=jnp.float32) + b[...]


_lin0 = pl.pallas_call(
    _lin0_body,
    grid=(_GRID,),
    in_specs=[_row_spec(_BLK, _H), _full_spec(_H, _H), _full_spec(1, _H)],
    out_specs=_row_spec(_BLK, _H),
    out_shape=jax.ShapeDtypeStruct((_N, _H), jnp.float32),
)


def _preb_body(hm, w2, b2, w3b, o):
    bb = jax.nn.relu(jnp.dot(hm[...], w2[...], preferred_element_type=jnp.float32)
                     + b2[...])
    o[...] = jnp.dot(bb, w3b[...], preferred_element_type=jnp.float32)


_preb = pl.pallas_call(
    _preb_body,
    grid=(_GRID,),
    in_specs=[
        _row_spec(_BLK, _H),
        _full_spec(_H, _H), _full_spec(1, _H), _full_spec(_H, 1),
    ],
    out_specs=_row_spec(_BLK, 1),
    out_shape=jax.ShapeDtypeStruct((_N, 1), jnp.float32),
)


def _mask_body(part, h, preb, w1, b1, w3a, b3, mask_o, hm2_o):
    aggr = part[0] + part[1]
    a = jax.nn.relu(jnp.dot(aggr, w1[...], preferred_element_type=jnp.float32)
                    + b1[...])
    logit = (jnp.dot(a, w3a[...], preferred_element_type=jnp.float32)
             + preb[...] + b3[...])
    mask = jax.nn.sigmoid(logit)
    mask_o[...] = mask
    hm2_o[...] = h[...] * mask


_mask_stage = pl.pallas_call(
    _mask_body,
    grid=(_GRID,),
    in_specs=[
        pl.BlockSpec((_NC, _BLK, _H), lambda i: (0, i, 0)),
        _row_spec(_BLK, _H),
        _row_spec(_BLK, 1),
        _full_spec(_H, _H), _full_spec(1, _H),
        _full_spec(_H, 1), _full_spec(1, 1),
    ],
    out_specs=[_row_spec(_BLK, 1), _row_spec(_BLK, _H)],
    out_shape=[
        jax.ShapeDtypeStruct((_N, 1), jnp.float32),
        jax.ShapeDtypeStruct((_N, _H), jnp.float32),
    ],
)


def _prec_body(hm2, w2, b2, o):
    o[...] = jnp.dot(hm2[...], w2[...],
                     preferred_element_type=jnp.float32) + b2[...]


_prec = pl.pallas_call(
    _prec_body,
    grid=(_GRID,),
    in_specs=[_row_spec(_BLK, _H), _full_spec(_H, _H), _full_spec(1, _H)],
    out_specs=_row_spec(_BLK, _H),
    out_shape=jax.ShapeDtypeStruct((_N, _H), jnp.float32),
)


def _conv_body(part, prec, mask, w1, b1, h_o, hmn_o):
    aggr = part[0] + part[1]
    h_new = jax.nn.relu(
        jnp.dot(aggr, w1[...], preferred_element_type=jnp.float32) + b1[...]
        + prec[...])
    h_o[...] = h_new
    hmn_o[...] = h_new * mask[...]


_conv_stage = pl.pallas_call(
    _conv_body,
    grid=(_GRID,),
    in_specs=[
        pl.BlockSpec((_NC, _BLK, _H), lambda i: (0, i, 0)),
        _row_spec(_BLK, _H),
        _row_spec(_BLK, 1),
        _full_spec(_H, _H), _full_spec(1, _H),
    ],
    out_specs=[_row_spec(_BLK, _H), _row_spec(_BLK, _H)],
    out_shape=[
        jax.ShapeDtypeStruct((_N, _H), jnp.float32),
        jax.ShapeDtypeStruct((_N, _H), jnp.float32),
    ],
)


def _head_body(h, batch, w1, b1, w2, b2, o, acc):
    i = pl.program_id(0)

    @pl.when(i == 0)
    def _():
        acc[...] = jnp.zeros_like(acc)

    onehot = (lax.broadcasted_iota(jnp.int32, (_B, _BLK), 0)
              == batch[0]).astype(jnp.float32)
    acc[...] += jnp.dot(onehot, h[...], preferred_element_type=jnp.float32)

    @pl.when(i == _GRID - 1)
    def _():
        pooled = acc[...]
        t = jax.nn.relu(
            jnp.dot(pooled, w1[...], preferred_element_type=jnp.float32) + b1[...])
        z = jnp.dot(t, w2[...], preferred_element_type=jnp.float32) + b2[...]
        m = jnp.max(z, axis=-1, keepdims=True)
        lse = m + jnp.log(jnp.sum(jnp.exp(z - m), axis=-1, keepdims=True))
        o[...] = z - lse


_head = pl.pallas_call(
    _head_body,
    grid=(_GRID,),
    in_specs=[
        _row_spec(_BLK, _H),
        pl.BlockSpec((1, 1, _BLK), lambda i: (i, 0, 0)),
        _full_spec(_H, _H), _full_spec(1, _H),
        _full_spec(_H, _C), _full_spec(1, _C),
    ],
    out_specs=_full_spec(_B, _C),
    out_shape=jax.ShapeDtypeStruct((_B, _C), jnp.float32),
    scratch_shapes=[pltpu.VMEM((_B, _H), jnp.float32)],
)


def kernel(x, edge_index, batch, lin0_W, lin0_b, mask_W1, mask_b1, mask_W2,
           mask_b2, mask_W3, mask_b3, conv_W1, conv_b1, conv_W2, conv_b2,
           lin1_W, lin1_b, lin2_W, lin2_b):
    pad = _EPWP - _EPW
    srcr = jnp.pad(edge_index[0].reshape(_NW, _EPW), ((0, 0), (0, pad)))
    dummy = _N + (jnp.arange(_NW, dtype=jnp.int32) % 8)[:, None]
    dstr = jnp.concatenate(
        [edge_index[1].reshape(_NW, _EPW),
         jnp.broadcast_to(dummy, (_NW, pad))],
        axis=1).reshape(_NW, _NCHUNK, _K)
    zrows = jnp.zeros((_ZROWS, _H), jnp.float32)
    batch3d = batch.reshape(_GRID, 1, _BLK)

    seg = _get_seg_kernel()
    h = _lin0(x, lin0_W, lin0_b.reshape(1, _H))
    hm = h
    for i in range(_L):
        part = seg(hm, srcr, dstr, zrows)
        preb = _preb(hm, mask_W2[i], mask_b2[i].reshape(1, _H),
                     mask_W3[i, _H:])
        mask, hm2 = _mask_stage(
            part, h, preb,
            mask_W1[i], mask_b1[i].reshape(1, _H),
            mask_W3[i, :_H], mask_b3[i].reshape(1, 1))
        part2 = seg(hm2, srcr, dstr, zrows)
        prec = _prec(hm2, conv_W2[i], conv_b2[i].reshape(1, _H))
        h, hm = _conv_stage(
            part2, prec, mask,
            conv_W1[i], conv_b1[i].reshape(1, _H))

    return _head(h, batch3d, lin1_W, lin1_b.reshape(1, _H),
                 lin2_W, lin2_b.reshape(1, _C))


# speedup vs baseline: 2.5482x; 1.6179x over previous
import functools

import jax
import jax.numpy as jnp
from jax import lax
from jax.experimental import pallas as pl
from jax.experimental.pallas import tpu as pltpu
from jax.experimental.pallas import tpu_sc as plsc

_N = 10000
_E = 320000
_H = 128
_B = 16
_C = 10
_L = 3

_NC = 2
_NS = 16
_NW = _NC * _NS
_EPW = _E // _NW
_K = 80
_NCHUNK = _EPW // _K

_ZROWS = 640
_ZLAST = _N - 15 * _ZROWS


def _seg_body(hm, srcr, dstr, zrows, part, src_v, dst_v, rows0, rows1,
              accum, gs0, gs1, ss0, ss1):
    cid = lax.axis_index("c")
    sid = lax.axis_index("s")
    wid = sid * _NC + cid

    pltpu.sync_copy(srcr.at[wid], src_v)
    pltpu.sync_copy(dstr.at[wid], dst_v)

    base = sid * _ZROWS

    @pl.when(sid < _NS - 1)
    def _():
        pltpu.sync_copy(zrows, accum.at[pl.ds(base, _ZROWS)])

    @pl.when(sid == _NS - 1)
    def _():
        pltpu.sync_copy(zrows.at[pl.ds(0, _ZLAST)],
                        accum.at[pl.ds(15 * _ZROWS, _ZLAST)])

    plsc.subcore_barrier()

    def gather(j, buf, sem):
        pltpu.async_copy(hm.at[src_v.at[pl.ds(j * _K, _K)]], buf, sem)

    def gwait(buf, sem):
        pltpu.make_async_copy(hm.at[src_v.at[pl.ds(0, _K)]], buf, sem).wait()

    def scat(j, buf, sem):
        pltpu.async_copy(buf, accum.at[dst_v.at[j]], sem, add=True)

    def swait(buf, sem):
        pltpu.make_async_copy(hm.at[src_v.at[pl.ds(0, _K)]], buf, sem).wait()

    gather(0, rows0, gs0)
    gather(1, rows1, gs1)

    def pipe(t, carry):
        j = 2 * t
        gwait(rows0, gs0)
        scat(j, rows0, ss0)

        @pl.when(j + 2 < _NCHUNK)
        def _():
            swait(rows0, ss0)
            gather(j + 2, rows0, gs0)

        @pl.when(j + 1 < _NCHUNK)
        def _():
            gwait(rows1, gs1)
            scat(j + 1, rows1, ss1)

        @pl.when(j + 3 < _NCHUNK)
        def _():
            swait(rows1, ss1)
            gather(j + 3, rows1, gs1)

        return carry

    lax.fori_loop(0, (_NCHUNK + 1) // 2, pipe, 0)
    swait(rows0, ss0)
    swait(rows1, ss1)

    plsc.subcore_barrier()

    @pl.when(sid < _NS - 1)
    def _():
        pltpu.sync_copy(accum.at[pl.ds(base, _ZROWS)],
                        part.at[cid, pl.ds(base, _ZROWS)])

    @pl.when(sid == _NS - 1)
    def _():
        pltpu.sync_copy(accum.at[pl.ds(15 * _ZROWS, _ZLAST)],
                        part.at[cid, pl.ds(15 * _ZROWS, _ZLAST)])


@functools.cache
def _get_seg_kernel():
    return pl.kernel(
        _seg_body,
        out_type=jax.ShapeDtypeStruct((_NC, _N, _H), jnp.float32),
        mesh=plsc.VectorSubcoreMesh(core_axis_name="c", subcore_axis_name="s",
                                    num_cores=_NC, num_subcores=_NS),
        scratch_types=[
            pltpu.VMEM((_EPW,), jnp.int32),
            pltpu.VMEM((_NCHUNK, _K), jnp.int32),
            pltpu.VMEM((_K, _H), jnp.float32),
            pltpu.VMEM((_K, _H), jnp.float32),
            pltpu.VMEM_SHARED((_N, _H), jnp.float32),
            pltpu.SemaphoreType.DMA,
            pltpu.SemaphoreType.DMA,
            pltpu.SemaphoreType.DMA,
            pltpu.SemaphoreType.DMA,
        ],
    )

_BLK = 1000
_GRID = _N // _BLK


def _row_spec(r, c):
    return pl.BlockSpec((r, c), lambda i: (i, 0))


def _full_spec(r, c):
    return pl.BlockSpec((r, c), lambda i: (0, 0))


def _lin0_body(x, w, b, o):
    o[...] = jnp.dot(x[...], w[...], preferred_element_type=jnp.float32) + b[...]


_lin0 = pl.pallas_call(
    _lin0_body,
    grid=(_GRID,),
    in_specs=[_row_spec(_BLK, _H), _full_spec(_H, _H), _full_spec(1, _H)],
    out_specs=_row_spec(_BLK, _H),
    out_shape=jax.ShapeDtypeStruct((_N, _H), jnp.float32),
)


def _mask_body(part, hm, h, w1, b1, w2, b2, w3a, w3b, b3, mask_o, hm2_o):
    aggr = part[0] + part[1]
    a = jax.nn.relu(jnp.dot(aggr, w1[...], preferred_element_type=jnp.float32)
                    + b1[...])
    bb = jax.nn.relu(jnp.dot(hm[...], w2[...], preferred_element_type=jnp.float32)
                     + b2[...])
    logit = (jnp.dot(a, w3a[...], preferred_element_type=jnp.float32)
             + jnp.dot(bb, w3b[...], preferred_element_type=jnp.float32)
             + b3[...])
    mask = jax.nn.sigmoid(logit)
    mask_o[...] = mask
    hm2_o[...] = h[...] * mask


_mask_stage = pl.pallas_call(
    _mask_body,
    grid=(_GRID,),
    in_specs=[
        pl.BlockSpec((_NC, _BLK, _H), lambda i: (0, i, 0)),
        _row_spec(_BLK, _H),
        _row_spec(_BLK, _H),
        _full_spec(_H, _H), _full_spec(1, _H),
        _full_spec(_H, _H), _full_spec(1, _H),
        _full_spec(_H, 1), _full_spec(_H, 1), _full_spec(1, 1),
    ],
    out_specs=[_row_spec(_BLK, 1), _row_spec(_BLK, _H)],
    out_shape=[
        jax.ShapeDtypeStruct((_N, 1), jnp.float32),
        jax.ShapeDtypeStruct((_N, _H), jnp.float32),
    ],
)


def _conv_body(part, hm2, mask, w1, b1, w2, b2, h_o, hmn_o):
    aggr = part[0] + part[1]
    h_new = jax.nn.relu(
        jnp.dot(aggr, w1[...], preferred_element_type=jnp.float32) + b1[...]
        + jnp.dot(hm2[...], w2[...], preferred_element_type=jnp.float32) + b2[...])
    h_o[...] = h_new
    hmn_o[...] = h_new * mask[...]


_conv_stage = pl.pallas_call(
    _conv_body,
    grid=(_GRID,),
    in_specs=[
        pl.BlockSpec((_NC, _BLK, _H), lambda i: (0, i, 0)),
        _row_spec(_BLK, _H),
        _row_spec(_BLK, 1),
        _full_spec(_H, _H), _full_spec(1, _H),
        _full_spec(_H, _H), _full_spec(1, _H),
    ],
    out_specs=[_row_spec(_BLK, _H), _row_spec(_BLK, _H)],
    out_shape=[
        jax.ShapeDtypeStruct((_N, _H), jnp.float32),
        jax.ShapeDtypeStruct((_N, _H), jnp.float32),
    ],
)


def _head_body(h, batch, w1, b1, w2, b2, o, acc):
    i = pl.program_id(0)

    @pl.when(i == 0)
    def _():
        acc[...] = jnp.zeros_like(acc)

    onehot = (lax.broadcasted_iota(jnp.int32, (_B, _BLK), 0)
              == batch[0]).astype(jnp.float32)
    acc[...] += jnp.dot(onehot, h[...], preferred_element_type=jnp.float32)

    @pl.when(i == _GRID - 1)
    def _():
        pooled = acc[...]
        t = jax.nn.relu(
            jnp.dot(pooled, w1[...], preferred_element_type=jnp.float32) + b1[...])
        z = jnp.dot(t, w2[...], preferred_element_type=jnp.float32) + b2[...]
        m = jnp.max(z, axis=-1, keepdims=True)
        lse = m + jnp.log(jnp.sum(jnp.exp(z - m), axis=-1, keepdims=True))
        o[...] = z - lse


_head = pl.pallas_call(
    _head_body,
    grid=(_GRID,),
    in_specs=[
        _row_spec(_BLK, _H),
        pl.BlockSpec((1, 1, _BLK), lambda i: (i, 0, 0)),
        _full_spec(_H, _H), _full_spec(1, _H),
        _full_spec(_H, _C), _full_spec(1, _C),
    ],
    out_specs=_full_spec(_B, _C),
    out_shape=jax.ShapeDtypeStruct((_B, _C), jnp.float32),
    scratch_shapes=[pltpu.VMEM((_B, _H), jnp.float32)],
)


def kernel(x, edge_index, batch, lin0_W, lin0_b, mask_W1, mask_b1, mask_W2,
           mask_b2, mask_W3, mask_b3, conv_W1, conv_b1, conv_W2, conv_b2,
           lin1_W, lin1_b, lin2_W, lin2_b):
    srcr = edge_index[0].reshape(_NW, _EPW)
    dstr = edge_index[1].reshape(_NW, _NCHUNK, _K)
    zrows = jnp.zeros((_ZROWS, _H), jnp.float32)
    batch3d = batch.reshape(_GRID, 1, _BLK)

    seg = _get_seg_kernel()
    h = _lin0(x, lin0_W, lin0_b.reshape(1, _H))
    hm = h
    for i in range(_L):
        part = seg(hm, srcr, dstr, zrows)
        mask, hm2 = _mask_stage(
            part, hm, h,
            mask_W1[i], mask_b1[i].reshape(1, _H),
            mask_W2[i], mask_b2[i].reshape(1, _H),
            mask_W3[i, :_H], mask_W3[i, _H:], mask_b3[i].reshape(1, 1))
        part2 = seg(hm2, srcr, dstr, zrows)
        h, hm = _conv_stage(
            part2, hm2, mask,
            conv_W1[i], conv_b1[i].reshape(1, _H),
            conv_W2[i], conv_b2[i].reshape(1, _H))

    return _head(h, batch3d, lin1_W, lin1_b.reshape(1, _H),
                 lin2_W, lin2_b.reshape(1, _C))


# speedup vs baseline: 2.5809x; 1.0129x over previous
import functools

import jax
import jax.numpy as jnp
from jax import lax
from jax.experimental import pallas as pl
from jax.experimental.pallas import tpu as pltpu
from jax.experimental.pallas import tpu_sc as plsc

_N = 10000
_E = 320000
_H = 128
_B = 16
_C = 10
_L = 3

_NC = 2
_NS = 16
_NW = _NC * _NS
_EPW = _E // _NW
_K = 80
_NCHUNK = _EPW // _K

_ZROWS = 640
_ZLAST = _N - 15 * _ZROWS


def _seg_body(hm, srcr, dstr, zrows, part, src_v, dst_v, rows0, rows1,
              accum, gs0, gs1, ss0, ss1):
    cid = lax.axis_index("c")
    sid = lax.axis_index("s")
    wid = sid * _NC + cid

    base = sid * _ZROWS

    pltpu.async_copy(srcr.at[wid], src_v, gs0)
    pltpu.async_copy(dstr.at[wid], dst_v, gs1)

    @pl.when(sid < _NS - 1)
    def _():
        pltpu.async_copy(zrows, accum.at[pl.ds(base, _ZROWS)], ss0)

    @pl.when(sid == _NS - 1)
    def _():
        pltpu.async_copy(zrows.at[pl.ds(0, _ZLAST)],
                         accum.at[pl.ds(15 * _ZROWS, _ZLAST)], ss0)

    def gather(j, buf, sem):
        pltpu.async_copy(hm.at[src_v.at[pl.ds(j * _K, _K)]], buf, sem)

    def gwait(buf, sem):
        pltpu.make_async_copy(hm.at[src_v.at[pl.ds(0, _K)]], buf, sem).wait()

    def scat(j, buf, sem):
        pltpu.async_copy(buf, accum.at[dst_v.at[j]], sem, add=True)

    def swait(buf, sem):
        pltpu.make_async_copy(hm.at[src_v.at[pl.ds(0, _K)]], buf, sem).wait()

    pltpu.make_async_copy(srcr.at[wid], src_v, gs0).wait()
    pltpu.make_async_copy(dstr.at[wid], dst_v, gs1).wait()
    gather(0, rows0, gs0)
    gather(1, rows1, gs1)

    @pl.when(sid < _NS - 1)
    def _():
        pltpu.make_async_copy(zrows, accum.at[pl.ds(base, _ZROWS)], ss0).wait()

    @pl.when(sid == _NS - 1)
    def _():
        pltpu.make_async_copy(zrows.at[pl.ds(0, _ZLAST)],
                              accum.at[pl.ds(15 * _ZROWS, _ZLAST)], ss0).wait()

    plsc.subcore_barrier()

    def pipe(t, carry):
        j = 2 * t
        gwait(rows0, gs0)
        scat(j, rows0, ss0)

        @pl.when(j + 2 < _NCHUNK)
        def _():
            swait(rows0, ss0)
            gather(j + 2, rows0, gs0)

        @pl.when(j + 1 < _NCHUNK)
        def _():
            gwait(rows1, gs1)
            scat(j + 1, rows1, ss1)

        @pl.when(j + 3 < _NCHUNK)
        def _():
            swait(rows1, ss1)
            gather(j + 3, rows1, gs1)

        return carry

    lax.fori_loop(0, (_NCHUNK + 1) // 2, pipe, 0)
    swait(rows0, ss0)
    swait(rows1, ss1)

    plsc.subcore_barrier()

    @pl.when(sid < _NS - 1)
    def _():
        pltpu.sync_copy(accum.at[pl.ds(base, _ZROWS)],
                        part.at[cid, pl.ds(base, _ZROWS)])

    @pl.when(sid == _NS - 1)
    def _():
        pltpu.sync_copy(accum.at[pl.ds(15 * _ZROWS, _ZLAST)],
                        part.at[cid, pl.ds(15 * _ZROWS, _ZLAST)])


@functools.cache
def _get_seg_kernel():
    return pl.kernel(
        _seg_body,
        out_type=jax.ShapeDtypeStruct((_NC, _N, _H), jnp.float32),
        mesh=plsc.VectorSubcoreMesh(core_axis_name="c", subcore_axis_name="s",
                                    num_cores=_NC, num_subcores=_NS),
        scratch_types=[
            pltpu.VMEM((_EPW,), jnp.int32),
            pltpu.VMEM((_NCHUNK, _K), jnp.int32),
            pltpu.VMEM((_K, _H), jnp.float32),
            pltpu.VMEM((_K, _H), jnp.float32),
            pltpu.VMEM_SHARED((_N, _H), jnp.float32),
            pltpu.SemaphoreType.DMA,
            pltpu.SemaphoreType.DMA,
            pltpu.SemaphoreType.DMA,
            pltpu.SemaphoreType.DMA,
        ],
    )

_BLK = 1000
_GRID = _N // _BLK


def _row_spec(r, c):
    return pl.BlockSpec((r, c), lambda i: (i, 0))


def _full_spec(r, c):
    return pl.BlockSpec((r, c), lambda i: (0, 0))


def _lin0_body(x, w, b, o):
    o[...] = jnp.dot(x[...], w[...], preferred_element_type=jnp.float32) + b[...]


_lin0 = pl.pallas_call(
    _lin0_body,
    grid=(_GRID,),
    in_specs=[_row_spec(_BLK, _H), _full_spec(_H, _H), _full_spec(1, _H)],
    out_specs=_row_spec(_BLK, _H),
    out_shape=jax.ShapeDtypeStruct((_N, _H), jnp.float32),
)


def _mask_body(part, hm, h, w1, b1, w2, b2, w3a, w3b, b3, mask_o, hm2_o):
    aggr = part[0] + part[1]
    a = jax.nn.relu(jnp.dot(aggr, w1[...], preferred_element_type=jnp.float32)
                    + b1[...])
    bb = jax.nn.relu(jnp.dot(hm[...], w2[...], preferred_element_type=jnp.float32)
                     + b2[...])
    logit = (jnp.dot(a, w3a[...], preferred_element_type=jnp.float32)
             + jnp.dot(bb, w3b[...], preferred_element_type=jnp.float32)
             + b3[...])
    mask = jax.nn.sigmoid(logit)
    mask_o[...] = mask
    hm2_o[...] = h[...] * mask


_mask_stage = pl.pallas_call(
    _mask_body,
    grid=(_GRID,),
    in_specs=[
        pl.BlockSpec((_NC, _BLK, _H), lambda i: (0, i, 0)),
        _row_spec(_BLK, _H),
        _row_spec(_BLK, _H),
        _full_spec(_H, _H), _full_spec(1, _H),
        _full_spec(_H, _H), _full_spec(1, _H),
        _full_spec(_H, 1), _full_spec(_H, 1), _full_spec(1, 1),
    ],
    out_specs=[_row_spec(_BLK, 1), _row_spec(_BLK, _H)],
    out_shape=[
        jax.ShapeDtypeStruct((_N, 1), jnp.float32),
        jax.ShapeDtypeStruct((_N, _H), jnp.float32),
    ],
)


def _conv_body(part, hm2, mask, w1, b1, w2, b2, h_o, hmn_o):
    aggr = part[0] + part[1]
    h_new = jax.nn.relu(
        jnp.dot(aggr, w1[...], preferred_element_type=jnp.float32) + b1[...]
        + jnp.dot(hm2[...], w2[...], preferred_element_type=jnp.float32) + b2[...])
    h_o[...] = h_new
    hmn_o[...] = h_new * mask[...]


_conv_stage = pl.pallas_call(
    _conv_body,
    grid=(_GRID,),
    in_specs=[
        pl.BlockSpec((_NC, _BLK, _H), lambda i: (0, i, 0)),
        _row_spec(_BLK, _H),
        _row_spec(_BLK, 1),
        _full_spec(_H, _H), _full_spec(1, _H),
        _full_spec(_H, _H), _full_spec(1, _H),
    ],
    out_specs=[_row_spec(_BLK, _H), _row_spec(_BLK, _H)],
    out_shape=[
        jax.ShapeDtypeStruct((_N, _H), jnp.float32),
        jax.ShapeDtypeStruct((_N, _H), jnp.float32),
    ],
)


def _head_body(h, batch, w1, b1, w2, b2, o, acc):
    i = pl.program_id(0)

    @pl.when(i == 0)
    def _():
        acc[...] = jnp.zeros_like(acc)

    onehot = (lax.broadcasted_iota(jnp.int32, (_B, _BLK), 0)
              == batch[0]).astype(jnp.float32)
    acc[...] += jnp.dot(onehot, h[...], preferred_element_type=jnp.float32)

    @pl.when(i == _GRID - 1)
    def _():
        pooled = acc[...]
        t = jax.nn.relu(
            jnp.dot(pooled, w1[...], preferred_element_type=jnp.float32) + b1[...])
        z = jnp.dot(t, w2[...], preferred_element_type=jnp.float32) + b2[...]
        m = jnp.max(z, axis=-1, keepdims=True)
        lse = m + jnp.log(jnp.sum(jnp.exp(z - m), axis=-1, keepdims=True))
        o[...] = z - lse


_head = pl.pallas_call(
    _head_body,
    grid=(_GRID,),
    in_specs=[
        _row_spec(_BLK, _H),
        pl.BlockSpec((1, 1, _BLK), lambda i: (i, 0, 0)),
        _full_spec(_H, _H), _full_spec(1, _H),
        _full_spec(_H, _C), _full_spec(1, _C),
    ],
    out_specs=_full_spec(_B, _C),
    out_shape=jax.ShapeDtypeStruct((_B, _C), jnp.float32),
    scratch_shapes=[pltpu.VMEM((_B, _H), jnp.float32)],
)


def kernel(x, edge_index, batch, lin0_W, lin0_b, mask_W1, mask_b1, mask_W2,
           mask_b2, mask_W3, mask_b3, conv_W1, conv_b1, conv_W2, conv_b2,
           lin1_W, lin1_b, lin2_W, lin2_b):
    srcr = edge_index[0].reshape(_NW, _EPW)
    dstr = edge_index[1].reshape(_NW, _NCHUNK, _K)
    zrows = jnp.zeros((_ZROWS, _H), jnp.float32)
    batch3d = batch.reshape(_GRID, 1, _BLK)

    seg = _get_seg_kernel()
    h = _lin0(x, lin0_W, lin0_b.reshape(1, _H))
    hm = h
    for i in range(_L):
        part = seg(hm, srcr, dstr, zrows)
        mask, hm2 = _mask_stage(
            part, hm, h,
            mask_W1[i], mask_b1[i].reshape(1, _H),
            mask_W2[i], mask_b2[i].reshape(1, _H),
            mask_W3[i, :_H], mask_W3[i, _H:], mask_b3[i].reshape(1, 1))
        part2 = seg(hm2, srcr, dstr, zrows)
        h, hm = _conv_stage(
            part2, hm2, mask,
            conv_W1[i], conv_b1[i].reshape(1, _H),
            conv_W2[i], conv_b2[i].reshape(1, _H))

    return _head(h, batch3d, lin1_W, lin1_b.reshape(1, _H),
                 lin2_W, lin2_b.reshape(1, _C))


# speedup vs baseline: 2.5870x; 1.0023x over previous
import functools

import jax
import jax.numpy as jnp
from jax import lax
from jax.experimental import pallas as pl
from jax.experimental.pallas import tpu as pltpu
from jax.experimental.pallas import tpu_sc as plsc

_N = 10000
_E = 320000
_H = 128
_B = 16
_C = 10
_L = 3

_NC = 2
_NS = 16
_NW = _NC * _NS
_EPW = _E // _NW
_K = 80
_NCHUNK = _EPW // _K

_ZROWS = 640
_ZLAST = _N - 15 * _ZROWS


def _seg_body(hm, srcr, dstr, zrows, part, src_v, dst_v, rows0, rows1,
              accum, gs0, gs1, ss0, ss1):
    cid = lax.axis_index("c")
    sid = lax.axis_index("s")
    wid = sid * _NC + cid

    base = sid * _ZROWS

    pltpu.async_copy(srcr.at[wid], src_v, gs0)
    pltpu.async_copy(dstr.at[wid], dst_v, gs1)

    @pl.when(sid < _NS - 1)
    def _():
        pltpu.async_copy(zrows, accum.at[pl.ds(base, _ZROWS)], ss0)

    @pl.when(sid == _NS - 1)
    def _():
        pltpu.async_copy(zrows.at[pl.ds(0, _ZLAST)],
                         accum.at[pl.ds(15 * _ZROWS, _ZLAST)], ss0)

    def gather(j, buf, sem):
        pltpu.async_copy(hm.at[src_v.at[pl.ds(j * _K, _K)]], buf, sem)

    def gwait(buf, sem):
        pltpu.make_async_copy(hm.at[src_v.at[pl.ds(0, _K)]], buf, sem).wait()

    def scat(j, buf, sem):
        pltpu.async_copy(buf, accum.at[dst_v.at[j]], sem, add=True)

    def swait(buf, sem):
        pltpu.make_async_copy(hm.at[src_v.at[pl.ds(0, _K)]], buf, sem).wait()

    pltpu.make_async_copy(srcr.at[wid], src_v, gs0).wait()
    pltpu.make_async_copy(dstr.at[wid], dst_v, gs1).wait()
    gather(0, rows0, gs0)
    gather(1, rows1, gs1)

    @pl.when(sid < _NS - 1)
    def _():
        pltpu.make_async_copy(zrows, accum.at[pl.ds(base, _ZROWS)], ss0).wait()

    @pl.when(sid == _NS - 1)
    def _():
        pltpu.make_async_copy(zrows.at[pl.ds(0, _ZLAST)],
                              accum.at[pl.ds(15 * _ZROWS, _ZLAST)], ss0).wait()

    plsc.subcore_barrier()

    def pipe(t, carry):
        j = 2 * t
        gwait(rows0, gs0)
        scat(j, rows0, ss0)
        swait(rows0, ss0)
        gather(j + 2, rows0, gs0)
        gwait(rows1, gs1)
        scat(j + 1, rows1, ss1)
        swait(rows1, ss1)
        gather(j + 3, rows1, gs1)
        return carry

    lax.fori_loop(0, (_NCHUNK - 3) // 2, pipe, 0)
    gwait(rows0, gs0)
    scat(_NCHUNK - 3, rows0, ss0)
    swait(rows0, ss0)
    gather(_NCHUNK - 1, rows0, gs0)
    gwait(rows1, gs1)
    scat(_NCHUNK - 2, rows1, ss1)
    gwait(rows0, gs0)
    scat(_NCHUNK - 1, rows0, ss0)
    swait(rows0, ss0)
    swait(rows1, ss1)

    plsc.subcore_barrier()

    @pl.when(sid < _NS - 1)
    def _():
        pltpu.sync_copy(accum.at[pl.ds(base, _ZROWS)],
                        part.at[cid, pl.ds(base, _ZROWS)])

    @pl.when(sid == _NS - 1)
    def _():
        pltpu.sync_copy(accum.at[pl.ds(15 * _ZROWS, _ZLAST)],
                        part.at[cid, pl.ds(15 * _ZROWS, _ZLAST)])


@functools.cache
def _get_seg_kernel():
    return pl.kernel(
        _seg_body,
        out_type=jax.ShapeDtypeStruct((_NC, _N, _H), jnp.float32),
        mesh=plsc.VectorSubcoreMesh(core_axis_name="c", subcore_axis_name="s",
                                    num_cores=_NC, num_subcores=_NS),
        scratch_types=[
            pltpu.VMEM((_EPW,), jnp.int32),
            pltpu.VMEM((_NCHUNK, _K), jnp.int32),
            pltpu.VMEM((_K, _H), jnp.float32),
            pltpu.VMEM((_K, _H), jnp.float32),
            pltpu.VMEM_SHARED((_N, _H), jnp.float32),
            pltpu.SemaphoreType.DMA,
            pltpu.SemaphoreType.DMA,
            pltpu.SemaphoreType.DMA,
            pltpu.SemaphoreType.DMA,
        ],
    )

_BLK = 1000
_GRID = _N // _BLK


def _row_spec(r, c):
    return pl.BlockSpec((r, c), lambda i: (i, 0))


def _full_spec(r, c):
    return pl.BlockSpec((r, c), lambda i: (0, 0))


def _lin0_body(x, w, b, o):
    o[...] = jnp.dot(x[...], w[...], preferred_element_type=jnp.float32) + b[...]


_lin0 = pl.pallas_call(
    _lin0_body,
    grid=(_GRID,),
    in_specs=[_row_spec(_BLK, _H), _full_spec(_H, _H), _full_spec(1, _H)],
    out_specs=_row_spec(_BLK, _H),
    out_shape=jax.ShapeDtypeStruct((_N, _H), jnp.float32),
)


def _mask_body(part, hm, h, w1, b1, w2, b2, w3a, w3b, b3, mask_o, hm2_o):
    aggr = part[0] + part[1]
    a = jax.nn.relu(jnp.dot(aggr, w1[...], preferred_element_type=jnp.float32)
                    + b1[...])
    bb = jax.nn.relu(jnp.dot(hm[...], w2[...], preferred_element_type=jnp.float32)
                     + b2[...])
    logit = (jnp.dot(a, w3a[...], preferred_element_type=jnp.float32)
             + jnp.dot(bb, w3b[...], preferred_element_type=jnp.float32)
             + b3[...])
    mask = jax.nn.sigmoid(logit)
    mask_o[...] = mask
    hm2_o[...] = h[...] * mask


_mask_stage = pl.pallas_call(
    _mask_body,
    grid=(_GRID,),
    in_specs=[
        pl.BlockSpec((_NC, _BLK, _H), lambda i: (0, i, 0)),
        _row_spec(_BLK, _H),
        _row_spec(_BLK, _H),
        _full_spec(_H, _H), _full_spec(1, _H),
        _full_spec(_H, _H), _full_spec(1, _H),
        _full_spec(_H, 1), _full_spec(_H, 1), _full_spec(1, 1),
    ],
    out_specs=[_row_spec(_BLK, 1), _row_spec(_BLK, _H)],
    out_shape=[
        jax.ShapeDtypeStruct((_N, 1), jnp.float32),
        jax.ShapeDtypeStruct((_N, _H), jnp.float32),
    ],
)


def _conv_body(part, hm2, mask, w1, b1, w2, b2, h_o, hmn_o):
    aggr = part[0] + part[1]
    h_new = jax.nn.relu(
        jnp.dot(aggr, w1[...], preferred_element_type=jnp.float32) + b1[...]
        + jnp.dot(hm2[...], w2[...], preferred_element_type=jnp.float32) + b2[...])
    h_o[...] = h_new
    hmn_o[...] = h_new * mask[...]


_conv_stage = pl.pallas_call(
    _conv_body,
    grid=(_GRID,),
    in_specs=[
        pl.BlockSpec((_NC, _BLK, _H), lambda i: (0, i, 0)),
        _row_spec(_BLK, _H),
        _row_spec(_BLK, 1),
        _full_spec(_H, _H), _full_spec(1, _H),
        _full_spec(_H, _H), _full_spec(1, _H),
    ],
    out_specs=[_row_spec(_BLK, _H), _row_spec(_BLK, _H)],
    out_shape=[
        jax.ShapeDtypeStruct((_N, _H), jnp.float32),
        jax.ShapeDtypeStruct((_N, _H), jnp.float32),
    ],
)


def _head_body(h, batch, w1, b1, w2, b2, o, acc):
    i = pl.program_id(0)

    @pl.when(i == 0)
    def _():
        acc[...] = jnp.zeros_like(acc)

    onehot = (lax.broadcasted_iota(jnp.int32, (_B, _BLK), 0)
              == batch[0]).astype(jnp.float32)
    acc[...] += jnp.dot(onehot, h[...], preferred_element_type=jnp.float32)

    @pl.when(i == _GRID - 1)
    def _():
        pooled = acc[...]
        t = jax.nn.relu(
            jnp.dot(pooled, w1[...], preferred_element_type=jnp.float32) + b1[...])
        z = jnp.dot(t, w2[...], preferred_element_type=jnp.float32) + b2[...]
        m = jnp.max(z, axis=-1, keepdims=True)
        lse = m + jnp.log(jnp.sum(jnp.exp(z - m), axis=-1, keepdims=True))
        o[...] = z - lse


_head = pl.pallas_call(
    _head_body,
    grid=(_GRID,),
    in_specs=[
        _row_spec(_BLK, _H),
        pl.BlockSpec((1, 1, _BLK), lambda i: (i, 0, 0)),
        _full_spec(_H, _H), _full_spec(1, _H),
        _full_spec(_H, _C), _full_spec(1, _C),
    ],
    out_specs=_full_spec(_B, _C),
    out_shape=jax.ShapeDtypeStruct((_B, _C), jnp.float32),
    scratch_shapes=[pltpu.VMEM((_B, _H), jnp.float32)],
)


def kernel(x, edge_index, batch, lin0_W, lin0_b, mask_W1, mask_b1, mask_W2,
           mask_b2, mask_W3, mask_b3, conv_W1, conv_b1, conv_W2, conv_b2,
           lin1_W, lin1_b, lin2_W, lin2_b):
    srcr = edge_index[0].reshape(_NW, _EPW)
    dstr = edge_index[1].reshape(_NW, _NCHUNK, _K)
    zrows = jnp.zeros((_ZROWS, _H), jnp.float32)
    batch3d = batch.reshape(_GRID, 1, _BLK)

    seg = _get_seg_kernel()
    h = _lin0(x, lin0_W, lin0_b.reshape(1, _H))
    hm = h
    for i in range(_L):
        part = seg(hm, srcr, dstr, zrows)
        mask, hm2 = _mask_stage(
            part, hm, h,
            mask_W1[i], mask_b1[i].reshape(1, _H),
            mask_W2[i], mask_b2[i].reshape(1, _H),
            mask_W3[i, :_H], mask_W3[i, _H:], mask_b3[i].reshape(1, 1))
        part2 = seg(hm2, srcr, dstr, zrows)
        h, hm = _conv_stage(
            part2, hm2, mask,
            conv_W1[i], conv_b1[i].reshape(1, _H),
            conv_W2[i], conv_b2[i].reshape(1, _H))

    return _head(h, batch3d, lin1_W, lin1_b.reshape(1, _H),
                 lin2_W, lin2_b.reshape(1, _C))
